# baseline (speedup 1.0000x reference)
import functools
import math

import jax
import jax.numpy as jnp
from jax import lax
from jax.experimental import pallas as pl
from jax.experimental.pallas import tpu as pltpu
from jax.experimental.pallas import tpu_sc as plsc

D_MODEL = 64
SCALE = math.sqrt(D_MODEL)

_NC = 2
_NS = 16
_NW = _NC * _NS

_ROWS = 4096
_COLS = 200
_XPT = _ROWS // _NW
_CH0, _CH1 = 96, 104
_NSTEP = _XPT * 2
_NBUF = 4
_PRE = 2
_NGRP = _NSTEP // _NBUF


def _body(x_hbm, table_hbm, out_hbm, idx_v, rows, gsems, osems, i0):
    def chunk_params(s, b):
        il = s // 2
        if b % 2 == 0:
            return il, 0, _CH0
        return il, _CH0, _CH1

    def gather(s, b):
        il, j0, ln = chunk_params(s, b)
        return pltpu.make_async_copy(
            table_hbm.at[idx_v.at[il, pl.ds(j0, ln)]],
            rows[b].at[pl.ds(0, ln), :], gsems[b])

    def scatter(s, b):
        il, j0, ln = chunk_params(s, b)
        return pltpu.make_async_copy(
            rows[b].at[pl.ds(0, ln), :],
            out_hbm.at[i0 + il, pl.ds(j0, ln), :], osems[b])

    pltpu.sync_copy(x_hbm.at[pl.ds(i0, _XPT), :], idx_v)

    for s in range(_PRE):
        gather(s, s % _NBUF).start()

    def group(grp, carry):
        for b in range(_NBUF):
            s = grp * _NBUF + b
            s_pre = s + _PRE
            b_pre = (b + _PRE) % _NBUF

            @pl.when(s_pre < _NSTEP)
            def _():
                @pl.when(s_pre >= _NBUF)
                def _():
                    scatter(s_pre - _NBUF, b_pre).wait()
                gather(s_pre, b_pre).start()

            gather(s, b).wait()

            buf = rows[b]
            _, _, _ln = chunk_params(0, b)

            @plsc.parallel_loop(0, _ln, step=1, unroll=8)
            def _scale(r):
                for c in range(D_MODEL // 16):
                    sl = pl.ds(c * 16, 16)
                    buf[r, sl] = buf[r, sl] * SCALE

            scatter(s, b).start()
        return carry

    lax.fori_loop(0, _NGRP, group, 0)

    for b in range(_NBUF):
        scatter(_NSTEP - _NBUF + b, b).wait()


@functools.partial(
    pl.kernel,
    out_type=jax.ShapeDtypeStruct((_ROWS, _COLS, D_MODEL), jnp.float32),
    mesh=plsc.VectorSubcoreMesh(core_axis_name="c", subcore_axis_name="s"),
    compiler_params=pltpu.CompilerParams(use_tc_tiling_on_sc=False),
    scratch_types=[
        pltpu.VMEM((_XPT, _COLS), jnp.int32),
        [pltpu.VMEM((_CH1, D_MODEL), jnp.float32) for _ in range(_NBUF)],
        [pltpu.SemaphoreType.DMA for _ in range(_NBUF)],
        [pltpu.SemaphoreType.DMA for _ in range(_NBUF)],
    ],
)
def _emb_lookup(x_hbm, table_hbm, out_hbm, idx_v, rows, gsems, osems):
    wid = lax.axis_index("s") * _NC + lax.axis_index("c")
    _body(x_hbm, table_hbm, out_hbm, idx_v, rows, gsems, osems,
          wid * _XPT)


@jax.jit
def kernel(x, table):
    return _emb_lookup(x, table)
